# baseline (speedup 1.0000x reference)
import jax
import jax.numpy as jnp
from jax.experimental import pallas as pl

_MARGIN = 1.0
_C = 32
_N = 1024
_DIM = 256
_EPS = 1e-6


def _dot(a, b, dims):
    return jax.lax.dot_general(a, b, (dims, ((), ())),
                               preferred_element_type=jnp.float32)


def _loss_kernel(x_ref, w_ref, b_ref, pos_ref, neg_ref):
    x = x_ref[...]
    w = w_ref[...]
    b = b_ref[...]

    emb = _dot(x, w, ((1,), (1,))) + b

    mx = jnp.max(emb, axis=1, keepdims=True)
    colf = jax.lax.broadcasted_iota(jnp.int32, (_N, _C), 1).astype(jnp.float32)
    lblf = jnp.min(jnp.where(emb == mx, colf, float(_C)),
                   axis=1, keepdims=True)
    onehot = (colf == lblf).astype(jnp.float32)

    ones_1c = jnp.ones((1, _C), jnp.float32)
    ones_1n = jnp.ones((1, _N), jnp.float32)
    ones_n1 = jnp.ones((_N, 1), jnp.float32)
    ones_c1 = jnp.ones((_C, 1), jnp.float32)

    w_col = _dot(onehot, ones_n1, ((0,), (0,)))
    w_row = _dot(ones_1n, onehot, ((1,), (0,)))
    w_safe = jnp.where(w_col == 0.0, 1.0, w_col)

    sums = _dot(onehot, emb, ((0,), (0,)))
    means = sums / w_safe

    r_col = jnp.sum(means * means, axis=1, keepdims=True)
    s_col = jnp.sum(means, axis=1, keepdims=True)
    r_row = _dot(ones_1c, means * means, ((1,), (1,)))
    s_row = _dot(ones_1c, means, ((1,), (1,)))
    gm = _dot(means, means, ((1,), (1,)))
    d2 = r_col + r_row - 2.0 * gm + 2.0 * _EPS * (s_col - s_row) \
        + _C * _EPS * _EPS
    d2 = jnp.maximum(d2, 1e-12)
    ir = jax.lax.broadcasted_iota(jnp.int32, (_C, _C), 0)
    ic = jax.lax.broadcasted_iota(jnp.int32, (_C, _C), 1)
    valid = (w_col > 0.0) & (w_row > 0.0) & (ic > ir)
    min_d2 = jnp.min(jnp.where(valid, d2, 1e30), axis=(0, 1), keepdims=True)
    neg = jnp.maximum(_MARGIN - min_d2, 0.0)
    neg = neg * neg
    n_nonempty = jnp.sum((w_col > 0.0).astype(jnp.float32),
                         axis=(0, 1), keepdims=True)
    neg_ref[...] = jnp.where(n_nonempty > 1.0, neg, 0.0)

    expects = _dot(onehot, means, ((1,), (0,)))
    e2 = emb - expects
    e2sq = e2 * e2
    te = (2.0 * _EPS) * e2
    rc = _dot(e2sq + te, ones_c1, ((1,), (0,)))
    rr = _dot(e2sq - te, ones_c1, ((1,), (0,))) + _C * _EPS * _EPS
    a_aug = jnp.concatenate([-2.0 * e2, rc, ones_n1], axis=1)
    b_aug = jnp.concatenate([e2, ones_n1, rr], axis=1)
    d2p = _dot(a_aug, b_aug, ((1,), (1,)))
    dist = jnp.sqrt(jnp.maximum(d2p, 1e-12)).astype(jnp.bfloat16)
    q = _dot(dist, onehot.astype(jnp.bfloat16), ((1,), (0,)))
    picked = jnp.sum(onehot * q, axis=1, keepdims=True)

    w2 = w_col - 1.0
    inv_w3 = 1.0 / jnp.where(w2 <= 0.0, 1.0, w2)
    a_s = _dot(onehot, inv_w3, ((1,), (0,)))
    pos_ref[...] = _dot(ones_1n, a_s * picked, ((1,), (0,))) / _C


def kernel(embeddings, W, b):
    pos, neg = pl.pallas_call(
        _loss_kernel,
        out_shape=(
            jax.ShapeDtypeStruct((1, 1), jnp.float32),
            jax.ShapeDtypeStruct((1, 1), jnp.float32),
        ),
    )(embeddings, W, b.reshape(1, _C))
    return pos[0, 0], neg[0, 0]


# speedup vs baseline: 1.0322x; 1.0322x over previous
import jax
import jax.numpy as jnp
from jax.experimental import pallas as pl

_MARGIN = 1.0
_C = 32
_N = 1024
_DIM = 256
_EPS = 1e-6
_BLK = 256


def _dot(a, b, dims):
    return jax.lax.dot_general(a, b, (dims, ((), ())),
                               preferred_element_type=jnp.float32)


def _loss_kernel(x_ref, w_ref, b_ref, pos_ref, neg_ref):
    x = x_ref[...]
    w = w_ref[...]
    b = b_ref[...]

    emb = _dot(x, w, ((1,), (1,))) + b

    mx = jnp.max(emb, axis=1, keepdims=True)
    colf = jax.lax.broadcasted_iota(jnp.int32, (_N, _C), 1).astype(jnp.float32)
    lblf = jnp.min(jnp.where(emb == mx, colf, float(_C)),
                   axis=1, keepdims=True)
    onehot = (colf == lblf).astype(jnp.float32)

    ones_1c = jnp.ones((1, _C), jnp.float32)
    ones_1n = jnp.ones((1, _N), jnp.float32)
    ones_n1 = jnp.ones((_N, 1), jnp.float32)
    ones_c1 = jnp.ones((_C, 1), jnp.float32)

    w_col = _dot(onehot, ones_n1, ((0,), (0,)))
    w_row = _dot(ones_1n, onehot, ((1,), (0,)))
    w_safe = jnp.where(w_col == 0.0, 1.0, w_col)

    sums = _dot(onehot, emb, ((0,), (0,)))
    means = sums / w_safe

    r_col = jnp.sum(means * means, axis=1, keepdims=True)
    s_col = jnp.sum(means, axis=1, keepdims=True)
    r_row = _dot(ones_1c, means * means, ((1,), (1,)))
    s_row = _dot(ones_1c, means, ((1,), (1,)))
    gm = _dot(means, means, ((1,), (1,)))
    d2 = r_col + r_row - 2.0 * gm + 2.0 * _EPS * (s_col - s_row) \
        + _C * _EPS * _EPS
    d2 = jnp.maximum(d2, 1e-12)
    ir = jax.lax.broadcasted_iota(jnp.int32, (_C, _C), 0)
    ic = jax.lax.broadcasted_iota(jnp.int32, (_C, _C), 1)
    valid = (w_col > 0.0) & (w_row > 0.0) & (ic > ir)
    min_d2 = jnp.min(jnp.where(valid, d2, 1e30), axis=(0, 1), keepdims=True)
    neg = jnp.maximum(_MARGIN - min_d2, 0.0)
    neg = neg * neg
    n_nonempty = jnp.sum((w_col > 0.0).astype(jnp.float32),
                         axis=(0, 1), keepdims=True)
    neg_ref[...] = jnp.where(n_nonempty > 1.0, neg, 0.0)

    expects = _dot(onehot, means, ((1,), (0,)))
    e2 = emb - expects
    e2sq = e2 * e2
    te = (2.0 * _EPS) * e2
    rc = _dot(e2sq + te, ones_c1, ((1,), (0,)))
    rr = _dot(e2sq - te, ones_c1, ((1,), (0,))) + _C * _EPS * _EPS
    a_aug = jnp.concatenate([-2.0 * e2, rc, ones_n1], axis=1)
    b_aug = jnp.concatenate([e2, ones_n1, rr], axis=1)

    nb = _N // _BLK
    q_acc = [None] * nb
    for rb in range(nb):
        ra = slice(rb * _BLK, (rb + 1) * _BLK)
        for cb in range(rb, nb):
            ca = slice(cb * _BLK, (cb + 1) * _BLK)
            d2p = _dot(a_aug[ra], b_aug[ca], ((1,), (1,)))
            dist = jnp.sqrt(jnp.maximum(d2p, 1e-12))
            qa = _dot(dist, onehot[ca], ((1,), (0,)))
            q_acc[rb] = qa if q_acc[rb] is None else q_acc[rb] + qa
            if cb != rb:
                qb = _dot(dist, onehot[ra], ((0,), (0,)))
                q_acc[cb] = qb if q_acc[cb] is None else q_acc[cb] + qb
    q = jnp.concatenate(q_acc, axis=0)
    picked = jnp.sum(onehot * q, axis=1, keepdims=True)

    w2 = w_col - 1.0
    inv_w3 = 1.0 / jnp.where(w2 <= 0.0, 1.0, w2)
    a_s = _dot(onehot, inv_w3, ((1,), (0,)))
    pos_ref[...] = _dot(ones_1n, a_s * picked, ((1,), (0,))) / _C


def kernel(embeddings, W, b):
    pos, neg = pl.pallas_call(
        _loss_kernel,
        out_shape=(
            jax.ShapeDtypeStruct((1, 1), jnp.float32),
            jax.ShapeDtypeStruct((1, 1), jnp.float32),
        ),
    )(embeddings, W, b.reshape(1, _C))
    return pos[0, 0], neg[0, 0]
